# speedup vs baseline: 3.5771x; 3.5771x over previous
import jax
import jax.numpy as jnp
from jax.experimental import pallas as pl

_B, _FL, _ML = 16, 2048, 256
_F, _H, _VE, _V = 128, 128, 32, 64
_NH, _DH, _DFF = 4, 32, 512


def _layer_norm(x, g, b):
    mu = jnp.mean(x, axis=-1, keepdims=True)
    d = x - mu
    var = jnp.mean(d * d, axis=-1, keepdims=True)
    return g * (d * jax.lax.rsqrt(var + 1e-5)) + b


def _body(vid_ref, feat_ref, mora_ref, emb_ref, Wpm_ref, bpm_ref, Wpf_ref,
          bpf_ref, Wq_ref, Wk_ref, Wv_ref, Wo_ref, ln1g_ref, ln1b_ref,
          W1_ref, b1_ref, W2_ref, b2_ref, ln2g_ref, ln2b_ref, Wpost_ref,
          bpost_ref, out_ref):
    feat = feat_ref[0]
    ids = mora_ref[0]
    ohT = (jax.lax.broadcasted_iota(jnp.int32, (_ML, _FL), 0) == ids
           ).astype(jnp.float32)
    cnt = jnp.sum(ohT, axis=1, keepdims=True)
    ssum = jnp.dot(ohT, feat, preferred_element_type=jnp.float32)
    inv = jnp.where(cnt > 0, 1.0 / jnp.maximum(cnt, 1.0), 0.0)
    mora_feat = ssum * inv

    vids = vid_ref[0]
    vohT = (jax.lax.broadcasted_iota(jnp.int32, (_V, _ML), 0) == vids
            ).astype(jnp.float32)
    EW = jnp.dot(emb_ref[...], Wpm_ref[:_VE, :],
                 preferred_element_type=jnp.float32)
    mhA = jax.lax.dot_general(vohT, EW, (((0,), (0,)), ((), ())),
                              preferred_element_type=jnp.float32)
    mh = (mhA + jnp.dot(mora_feat, Wpm_ref[_VE:, :],
                        preferred_element_type=jnp.float32)
          + bpm_ref[...])
    fh = jnp.dot(feat, Wpf_ref[...],
                 preferred_element_type=jnp.float32) + bpf_ref[...]

    q = jnp.dot(mh, Wq_ref[...], preferred_element_type=jnp.float32)
    k = jnp.dot(fh, Wk_ref[...], preferred_element_type=jnp.float32)
    v = jnp.dot(fh, Wv_ref[...], preferred_element_type=jnp.float32)

    scale = 1.0 / (_DH ** 0.5)
    ctxs = []
    for h_i in range(_NH):
        sl = slice(h_i * _DH, (h_i + 1) * _DH)
        qi = q[:, sl]
        ki = k[:, sl]
        vi = v[:, sl]
        s = jax.lax.dot_general(qi, ki, (((1,), (1,)), ((), ())),
                                preferred_element_type=jnp.float32) * scale
        m = jnp.max(s, axis=1, keepdims=True)
        e = jnp.exp(s - m)
        den = jnp.sum(e, axis=1, keepdims=True)
        p = e / den
        ctxs.append(jnp.dot(p, vi, preferred_element_type=jnp.float32))
    ctx = jnp.concatenate(ctxs, axis=1)

    h = mh + jnp.dot(ctx, Wo_ref[...], preferred_element_type=jnp.float32)
    h = _layer_norm(h, ln1g_ref[...], ln1b_ref[...])
    ff = jnp.maximum(
        jnp.dot(h, W1_ref[...], preferred_element_type=jnp.float32)
        + b1_ref[...], 0.0)
    h2 = h + jnp.dot(ff, W2_ref[...],
                     preferred_element_type=jnp.float32) + b2_ref[...]
    h2 = _layer_norm(h2, ln2g_ref[...], ln2b_ref[...])
    out_ref[0] = jnp.dot(h2, Wpost_ref[...],
                         preferred_element_type=jnp.float32) + bpost_ref[...]


def kernel(vowel_ids, features, mora_index, emb, Wpm, bpm, Wpf, bpf, Wq, Wk,
           Wv, Wo, ln1_g, ln1_b, W1, b1, W2, b2, ln2_g, ln2_b, Wpost, bpost):
    B_, FL_, F_ = features.shape
    ML_ = vowel_ids.shape[1]

    vid3 = vowel_ids.astype(jnp.int32).reshape(B_, 1, ML_)
    mora3 = mora_index.astype(jnp.int32).reshape(B_, 1, FL_)
    row = lambda x: x.reshape(1, -1)

    def full(arr):
        return pl.BlockSpec(arr.shape, lambda b: (0,) * arr.ndim)

    weights = [emb, Wpm, row(bpm), Wpf, row(bpf), Wq, Wk, Wv, Wo,
               row(ln1_g), row(ln1_b), W1, row(b1), W2, row(b2),
               row(ln2_g), row(ln2_b), Wpost, row(bpost)]

    out = pl.pallas_call(
        _body,
        grid=(B_,),
        in_specs=[
            pl.BlockSpec((1, 1, ML_), lambda b: (b, 0, 0)),
            pl.BlockSpec((1, FL_, F_), lambda b: (b, 0, 0)),
            pl.BlockSpec((1, 1, FL_), lambda b: (b, 0, 0)),
        ] + [full(w) for w in weights],
        out_specs=pl.BlockSpec((1, ML_, 8), lambda b: (b, 0, 0)),
        out_shape=jax.ShapeDtypeStruct((B_, ML_, 8), jnp.float32),
    )(vid3, features, mora3, *weights)
    return out.reshape(B_, ML_, 2, 4)


# speedup vs baseline: 4.2897x; 1.1992x over previous
import jax
import jax.numpy as jnp
from jax.experimental import pallas as pl

_B, _FL, _ML = 16, 2048, 256
_F, _H, _VE, _V = 128, 128, 32, 64
_NH, _DH, _DFF = 4, 32, 512
_BF = jnp.bfloat16


def _layer_norm(x, g, b):
    mu = jnp.mean(x, axis=-1, keepdims=True)
    d = x - mu
    var = jnp.mean(d * d, axis=-1, keepdims=True)
    return g * (d * jax.lax.rsqrt(var + 1e-5)) + b


def _bdot(a, b):
    return jnp.dot(a.astype(_BF), b.astype(_BF),
                   preferred_element_type=jnp.float32)


def _body(vid_ref, feat_ref, mora_ref, emb_ref, Wpm_ref, bpm_ref, Wpf_ref,
          bpf_ref, Wq_ref, Wk_ref, Wv_ref, Wo_ref, ln1g_ref, ln1b_ref,
          W1_ref, b1_ref, W2_ref, b2_ref, ln2g_ref, ln2b_ref, Wpost_ref,
          bpost_ref, out_ref):
    feat = feat_ref[0].astype(_BF)
    ids = mora_ref[0]
    ohT = (jax.lax.broadcasted_iota(jnp.int32, (_ML, _FL), 0) == ids
           ).astype(_BF)
    cnt = jnp.sum(ohT.astype(jnp.float32), axis=1, keepdims=True)
    ssum = jnp.dot(ohT, feat, preferred_element_type=jnp.float32)
    inv = jnp.where(cnt > 0, 1.0 / jnp.maximum(cnt, 1.0), 0.0)
    mora_feat = ssum * inv

    vids = vid_ref[0]
    vohT = (jax.lax.broadcasted_iota(jnp.int32, (_V, _ML), 0) == vids
            ).astype(_BF)
    EW = _bdot(emb_ref[...], Wpm_ref[:_VE, :])
    mhA = jax.lax.dot_general(vohT, EW.astype(_BF), (((0,), (0,)), ((), ())),
                              preferred_element_type=jnp.float32)
    mh = mhA + _bdot(mora_feat, Wpm_ref[_VE:, :]) + bpm_ref[...]

    Wk_eff = _bdot(Wpf_ref[...], Wk_ref[...]).astype(_BF)
    Wv_eff = _bdot(Wpf_ref[...], Wv_ref[...]).astype(_BF)
    bv = _bdot(bpf_ref[...], Wv_ref[...])
    k = jnp.dot(feat, Wk_eff, preferred_element_type=jnp.float32)
    v = jnp.dot(feat, Wv_eff, preferred_element_type=jnp.float32)
    q = _bdot(mh, Wq_ref[...])

    scale = 1.0 / (_DH ** 0.5)
    ones_col = jnp.ones((_FL, 8), dtype=_BF)
    ctxs = []
    for h_i in range(_NH):
        sl = slice(h_i * _DH, (h_i + 1) * _DH)
        qi = (q[:, sl] * scale).astype(_BF)
        ki = k[:, sl].astype(_BF)
        vi = v[:, sl].astype(_BF)
        s = jax.lax.dot_general(qi, ki, (((1,), (1,)), ((), ())),
                                preferred_element_type=jnp.float32)
        m = jnp.max(s, axis=1, keepdims=True)
        e = jnp.exp(s - m).astype(_BF)
        viaug = jnp.concatenate([vi, ones_col], axis=1)
        cd = jnp.dot(e, viaug, preferred_element_type=jnp.float32)
        ctxs.append(cd[:, :_DH] / cd[:, _DH:_DH + 1])
    ctx = jnp.concatenate(ctxs, axis=1) + bv

    h = mh + _bdot(ctx, Wo_ref[...])
    h = _layer_norm(h, ln1g_ref[...], ln1b_ref[...])
    ff = jnp.maximum(_bdot(h, W1_ref[...]) + b1_ref[...], 0.0)
    h2 = h + _bdot(ff, W2_ref[...]) + b2_ref[...]
    h2 = _layer_norm(h2, ln2g_ref[...], ln2b_ref[...])
    out_ref[0] = _bdot(h2, Wpost_ref[...]) + bpost_ref[...]


def kernel(vowel_ids, features, mora_index, emb, Wpm, bpm, Wpf, bpf, Wq, Wk,
           Wv, Wo, ln1_g, ln1_b, W1, b1, W2, b2, ln2_g, ln2_b, Wpost, bpost):
    B_, FL_, F_ = features.shape
    ML_ = vowel_ids.shape[1]

    vid3 = vowel_ids.astype(jnp.int32).reshape(B_, 1, ML_)
    mora3 = mora_index.astype(jnp.int32).reshape(B_, 1, FL_)
    row = lambda x: x.reshape(1, -1)

    def full(arr):
        return pl.BlockSpec(arr.shape, lambda b: (0,) * arr.ndim)

    weights = [emb, Wpm, row(bpm), Wpf, row(bpf), Wq, Wk, Wv, Wo,
               row(ln1_g), row(ln1_b), W1, row(b1), W2, row(b2),
               row(ln2_g), row(ln2_b), Wpost, row(bpost)]

    out = pl.pallas_call(
        _body,
        grid=(B_,),
        in_specs=[
            pl.BlockSpec((1, 1, ML_), lambda b: (b, 0, 0)),
            pl.BlockSpec((1, FL_, F_), lambda b: (b, 0, 0)),
            pl.BlockSpec((1, 1, FL_), lambda b: (b, 0, 0)),
        ] + [full(w) for w in weights],
        out_specs=pl.BlockSpec((1, ML_, 8), lambda b: (b, 0, 0)),
        out_shape=jax.ShapeDtypeStruct((B_, ML_, 8), jnp.float32),
    )(vid3, features, mora3, *weights)
    return out.reshape(B_, ML_, 2, 4)


# speedup vs baseline: 4.9917x; 1.1636x over previous
import jax
import jax.numpy as jnp
from jax.experimental import pallas as pl

_B, _FL, _ML = 16, 2048, 256
_F, _H, _VE, _V = 128, 128, 32, 64
_NH, _DH, _DFF = 4, 32, 512
_BF = jnp.bfloat16


def _layer_norm(x, g, b):
    mu = jnp.mean(x, axis=-1, keepdims=True)
    d = x - mu
    var = jnp.mean(d * d, axis=-1, keepdims=True)
    return g * (d * jax.lax.rsqrt(var + 1e-5)) + b


def _bdot(a, b):
    return jnp.dot(a.astype(_BF), b.astype(_BF),
                   preferred_element_type=jnp.float32)


def _body(vid_ref, feat_ref, mora_ref, emb_ref, Wpm_ref, bpm_ref, Wpf_ref,
          bpf_ref, Wq_ref, Wk_ref, Wv_ref, Wo_ref, ln1g_ref, ln1b_ref,
          W1_ref, b1_ref, W2_ref, b2_ref, ln2g_ref, ln2b_ref, Wpost_ref,
          bpost_ref, out_ref):
    feat = feat_ref[0].astype(_BF)
    ids = mora_ref[0]
    ohT = (jax.lax.broadcasted_iota(jnp.int32, (_ML, _FL), 0) == ids
           ).astype(_BF)
    cnt = jnp.sum(ohT.astype(jnp.float32), axis=1, keepdims=True)
    ssum = jnp.dot(ohT, feat, preferred_element_type=jnp.float32)
    inv = jnp.where(cnt > 0, 1.0 / jnp.maximum(cnt, 1.0), 0.0)
    mora_feat = ssum * inv

    vids = vid_ref[0]
    vohT = (jax.lax.broadcasted_iota(jnp.int32, (_V, _ML), 0) == vids
            ).astype(_BF)
    EW = _bdot(emb_ref[...], Wpm_ref[:_VE, :])
    mhA = jax.lax.dot_general(vohT, EW.astype(_BF), (((0,), (0,)), ((), ())),
                              preferred_element_type=jnp.float32)
    mh = mhA + _bdot(mora_feat, Wpm_ref[_VE:, :]) + bpm_ref[...]

    scale = 1.0 / (_DH ** 0.5)
    Wk_eff = _bdot(Wpf_ref[...], Wk_ref[...]).astype(_BF)
    Wv_eff = _bdot(Wpf_ref[...], Wv_ref[...]).astype(_BF)
    bv = _bdot(bpf_ref[...], Wv_ref[...])
    k = jnp.dot(feat, Wk_eff,
                preferred_element_type=jnp.float32).astype(_BF)
    v = jnp.dot(feat, Wv_eff,
                preferred_element_type=jnp.float32).astype(_BF)
    q = _bdot(mh, Wq_ref[...] * scale).astype(_BF)

    ones_col = jnp.ones((_FL, 8), dtype=_BF)
    ctxs = []
    for h_i in range(_NH):
        sl = slice(h_i * _DH, (h_i + 1) * _DH)
        s = jax.lax.dot_general(q[:, sl], k[:, sl], (((1,), (1,)), ((), ())),
                                preferred_element_type=jnp.float32)
        e = jnp.exp(s).astype(_BF)
        viaug = jnp.concatenate([v[:, sl], ones_col], axis=1)
        cd = jnp.dot(e, viaug, preferred_element_type=jnp.float32)
        ctxs.append(cd[:, :_DH] / cd[:, _DH:_DH + 1])
    ctx = jnp.concatenate(ctxs, axis=1) + bv

    h = mh + _bdot(ctx, Wo_ref[...])
    h = _layer_norm(h, ln1g_ref[...], ln1b_ref[...])
    ff = jnp.maximum(_bdot(h, W1_ref[...]) + b1_ref[...], 0.0)
    h2 = h + _bdot(ff, W2_ref[...]) + b2_ref[...]
    h2 = _layer_norm(h2, ln2g_ref[...], ln2b_ref[...])
    out_ref[0] = _bdot(h2, Wpost_ref[...]) + bpost_ref[...]


def kernel(vowel_ids, features, mora_index, emb, Wpm, bpm, Wpf, bpf, Wq, Wk,
           Wv, Wo, ln1_g, ln1_b, W1, b1, W2, b2, ln2_g, ln2_b, Wpost, bpost):
    B_, FL_, F_ = features.shape
    ML_ = vowel_ids.shape[1]

    vid3 = vowel_ids.astype(jnp.int32).reshape(B_, 1, ML_)
    mora3 = mora_index.astype(jnp.int32).reshape(B_, 1, FL_)
    row = lambda x: x.reshape(1, -1)

    def full(arr):
        return pl.BlockSpec(arr.shape, lambda b: (0,) * arr.ndim)

    weights = [emb, Wpm, row(bpm), Wpf, row(bpf), Wq, Wk, Wv, Wo,
               row(ln1_g), row(ln1_b), W1, row(b1), W2, row(b2),
               row(ln2_g), row(ln2_b), Wpost, row(bpost)]

    out = pl.pallas_call(
        _body,
        grid=(B_,),
        in_specs=[
            pl.BlockSpec((1, 1, ML_), lambda b: (b, 0, 0)),
            pl.BlockSpec((1, FL_, F_), lambda b: (b, 0, 0)),
            pl.BlockSpec((1, 1, FL_), lambda b: (b, 0, 0)),
        ] + [full(w) for w in weights],
        out_specs=pl.BlockSpec((1, ML_, 8), lambda b: (b, 0, 0)),
        out_shape=jax.ShapeDtypeStruct((B_, ML_, 8), jnp.float32),
    )(vid3, features, mora3, *weights)
    return out.reshape(B_, ML_, 2, 4)
